# baseline (speedup 1.0000x reference)
import functools

import jax
import jax.numpy as jnp
from jax import lax
from jax.experimental import pallas as pl
from jax.experimental.pallas import tpu as pltpu
from jax.experimental.pallas import tpu_sc as plsc

N = 10000
E = 160000
D_IN = 256
D_OUT = 256
H = 128
NT = 16
RPT = 632
RPT_LAST = N - (NT - 1) * RPT

_mesh = plsc.VectorSubcoreMesh(core_axis_name="c", subcore_axis_name="s")


def _fill_ones(ref, n):
    one16 = jnp.full((16,), 1.0, jnp.float32)
    for j in range(n // 16):
        ref[pl.ds(j * 16, 16)] = one16
    if n % 16:
        ref[pl.ds(n - 16, 16)] = one16


CHA = 40
EPTA = E // (2 * NT)
NCHA = EPTA // CHA
KA = 5
NGA = NCHA // KA


@functools.partial(
    pl.kernel,
    out_type=jax.ShapeDtypeStruct((2, N), jnp.float32),
    mesh=_mesh,
    scratch_types=[
        pltpu.VMEM((2, KA, CHA), jnp.int32),
        pltpu.VMEM((CHA,), jnp.float32),
        pltpu.VMEM_SHARED((N,), jnp.float32),
        pltpu.SemaphoreType.DMA((2,)),
        pltpu.SemaphoreType.DMA((2,)),
    ],
)
def _degree_kernel(src_hbm, zeros_hbm, deg_hbm, idx_v, ones_v, acc_sh,
                   sem_i, sem_s):
    c = lax.axis_index("c")
    s = lax.axis_index("s")

    @pl.when(s == 0)
    def _():
        pltpu.sync_copy(zeros_hbm, acc_sh)

    _fill_ones(ones_v, CHA)
    plsc.subcore_barrier()

    base = c * (E // 2) + s * EPTA

    def issue_group(g, p):
        for b in range(KA):
            off = base + g * (KA * CHA) + b * CHA
            pltpu.async_copy(src_hbm.at[pl.ds(off, CHA)],
                             idx_v.at[p, b], sem_i.at[p])

    def drain_scatters(p):
        for b in range(KA):
            pltpu.make_async_copy(ones_v, acc_sh.at[idx_v.at[p, b]],
                                  sem_s.at[p]).wait()

    issue_group(0, 0)

    def body(g, carry):
        p = lax.rem(g, 2)
        q = 1 - p

        @pl.when(g >= 1)
        def _():
            drain_scatters(q)

        @pl.when(g + 1 < NGA)
        def _():
            issue_group(g + 1, q)

        for b in range(KA):
            pltpu.make_async_copy(src_hbm.at[pl.ds(base, CHA)],
                                  idx_v.at[p, b], sem_i.at[p]).wait()
        for b in range(KA):
            pltpu.async_copy(ones_v, acc_sh.at[idx_v.at[p, b]],
                             sem_s.at[p], add=True)
        return carry

    lax.fori_loop(0, NGA, body, 0)
    drain_scatters((NGA - 1) % 2)

    plsc.subcore_barrier()

    @pl.when(s == 0)
    def _():
        pltpu.sync_copy(acc_sh, deg_hbm.at[c])


def _scale_body(x_ref, degt_ref, gl_ref, gr_ref):
    dt = degt_ref[...]
    d2 = lax.rsqrt(jnp.maximum(dt[:, 0:1] + dt[:, 1:2], 1.0))
    g = x_ref[...] * d2
    gl_ref[...] = g[:, :H]
    gr_ref[...] = g[:, H:]


def _scale_call(x, degT):
    R = 2000
    grid = N // R
    return pl.pallas_call(
        _scale_body,
        grid=(grid,),
        in_specs=[
            pl.BlockSpec((R, D_IN), lambda i: (i, 0)),
            pl.BlockSpec((R, 2), lambda i: (i, 0)),
        ],
        out_specs=[
            pl.BlockSpec((R, H), lambda i: (i, 0)),
            pl.BlockSpec((R, H), lambda i: (i, 0)),
        ],
        out_shape=[
            jax.ShapeDtypeStruct((N, H), jnp.float32),
            jax.ShapeDtypeStruct((N, H), jnp.float32),
        ],
    )(x, degT)


CHC = 40
EPT = E // NT
NCHUNKC = EPT // CHC
KC = 2
NGC = NCHUNKC // KC


@functools.partial(
    pl.kernel,
    out_type=[
        jax.ShapeDtypeStruct((N, H), jnp.float32),
        jax.ShapeDtypeStruct((N, H), jnp.float32),
        jax.ShapeDtypeStruct((N,), jnp.float32),
    ],
    mesh=_mesh,
    scratch_types=[
        pltpu.VMEM((EPT,), jnp.int32),
        pltpu.VMEM((2, KC, CHC), jnp.int32),
        pltpu.VMEM((2, KC, CHC, H), jnp.float32),
        pltpu.VMEM((CHC,), jnp.float32),
        pltpu.VMEM_SHARED((N, H), jnp.float32),
        pltpu.VMEM_SHARED((N,), jnp.float32),
        pltpu.SemaphoreType.DMA((2,)),
        pltpu.SemaphoreType.DMA((2,)),
        pltpu.SemaphoreType.DMA((2,)),
    ],
)
def _gather_scatter_kernel(gl_hbm, gr_hbm, src_hbm, dst_hbm, zeros_hbm,
                           zerosn_hbm, aggl_hbm, aggr_hbm, indeg_hbm,
                           src_all, dst_v, rows_v, ones_v, acc_sh, deg_sh,
                           sem_i, sem_g, sem_s):
    c = lax.axis_index("c")
    s = lax.axis_index("s")

    def _rowslice(ref):
        start = pl.multiple_of(s * RPT, 8)
        return ref.at[pl.ds(start, RPT)]

    def _rowslice_last(ref):
        return ref.at[pl.ds((NT - 1) * RPT, RPT_LAST)]

    @pl.when(s < NT - 1)
    def _():
        pltpu.sync_copy(_rowslice(zeros_hbm), _rowslice(acc_sh))

    @pl.when(s == NT - 1)
    def _():
        pltpu.sync_copy(_rowslice_last(zeros_hbm), _rowslice_last(acc_sh))

    @pl.when(jnp.logical_and(c == 0, s == 0))
    def _():
        pltpu.sync_copy(zerosn_hbm, deg_sh)

    _fill_ones(ones_v, CHC)

    plsc.subcore_barrier()

    base = s * EPT

    def run(g_hbm):
        pltpu.sync_copy(src_hbm.at[pl.ds(base, EPT)], src_all)

        def issue_group(g, p):
            for b in range(KC):
                off = g * (KC * CHC) + b * CHC
                pltpu.async_copy(dst_hbm.at[pl.ds(base + off, CHC)],
                                 dst_v.at[p, b], sem_i.at[p])
                pltpu.async_copy(g_hbm.at[src_all.at[pl.ds(off, CHC)]],
                                 rows_v.at[p, b], sem_g.at[p])

        def drain_scatters(p):
            for b in range(KC):
                pltpu.make_async_copy(rows_v.at[p, b],
                                      acc_sh.at[dst_v.at[p, b]],
                                      sem_s.at[p]).wait()

                @pl.when(c == 0)
                def _():
                    pltpu.make_async_copy(ones_v, deg_sh.at[dst_v.at[p, b]],
                                          sem_s.at[p]).wait()

        issue_group(0, 0)

        def body(g, carry):
            p = lax.rem(g, 2)
            q = 1 - p

            @pl.when(g >= 1)
            def _():
                drain_scatters(q)

            @pl.when(g + 1 < NGC)
            def _():
                issue_group(g + 1, q)

            for b in range(KC):
                pltpu.make_async_copy(g_hbm.at[src_all.at[pl.ds(0, CHC)]],
                                      rows_v.at[p, b], sem_g.at[p]).wait()
            for b in range(KC):
                pltpu.make_async_copy(dst_hbm.at[pl.ds(base, CHC)],
                                      dst_v.at[p, b], sem_i.at[p]).wait()
            for b in range(KC):
                pltpu.async_copy(rows_v.at[p, b], acc_sh.at[dst_v.at[p, b]],
                                 sem_s.at[p], add=True)

                @pl.when(c == 0)
                def _():
                    pltpu.async_copy(ones_v, deg_sh.at[dst_v.at[p, b]],
                                     sem_s.at[p], add=True)
            return carry

        lax.fori_loop(0, NGC, body, 0)
        drain_scatters((NGC - 1) % 2)

    @pl.when(c == 0)
    def _():
        run(gl_hbm)

    @pl.when(c == 1)
    def _():
        run(gr_hbm)

    plsc.subcore_barrier()

    def writeout(agg_hbm):
        @pl.when(s < NT - 1)
        def _():
            pltpu.sync_copy(_rowslice(acc_sh), _rowslice(agg_hbm))

        @pl.when(s == NT - 1)
        def _():
            pltpu.sync_copy(_rowslice_last(acc_sh), _rowslice_last(agg_hbm))

    @pl.when(c == 0)
    def _():
        writeout(aggl_hbm)

        @pl.when(s == 0)
        def _():
            pltpu.sync_copy(deg_sh, indeg_hbm)

    @pl.when(c == 1)
    def _():
        writeout(aggr_hbm)


def _matmul_body(al_ref, ar_ref, w_ref, ind_ref, ft_ref):
    w = w_ref[...]
    acc = lax.dot_general(al_ref[...], w[:, :H], (((1,), (1,)), ((), ())),
                          preferred_element_type=jnp.float32)
    acc = acc + lax.dot_general(ar_ref[...], w[:, H:], (((1,), (1,)), ((), ())),
                                preferred_element_type=jnp.float32)
    d0 = lax.rsqrt(jnp.maximum(ind_ref[...], 1.0))
    ft_ref[...] = acc * d0


def _matmul_call(aggl, aggr, W_fc, ind):
    R = 2000
    grid = N // R
    return pl.pallas_call(
        _matmul_body,
        grid=(grid,),
        in_specs=[
            pl.BlockSpec((R, H), lambda i: (i, 0)),
            pl.BlockSpec((R, H), lambda i: (i, 0)),
            pl.BlockSpec((D_OUT, D_IN), lambda i: (0, 0)),
            pl.BlockSpec((R, 1), lambda i: (i, 0)),
        ],
        out_specs=pl.BlockSpec((R, D_OUT), lambda i: (i, 0)),
        out_shape=jax.ShapeDtypeStruct((N, D_OUT), jnp.float32),
    )(aggl, aggr, W_fc, ind)


def kernel(x, loc, edge_index, inter_ids, W_fc, W_G, embed_table, boundaries):
    src = edge_index[0]
    dst = edge_index[1]
    zeros_n = jnp.zeros((N,), jnp.float32)
    zeros_nd = jnp.zeros((N, H), jnp.float32)

    deg = _degree_kernel(src, zeros_n)
    gl, gr = _scale_call(x, deg.T)
    aggl, aggr, indeg = _gather_scatter_kernel(gl, gr, src, dst,
                                               zeros_nd, zeros_n)
    return _matmul_call(aggl, aggr, W_fc, indeg.reshape(N, 1))


# speedup vs baseline: 1.1596x; 1.1596x over previous
import functools

import jax
import jax.numpy as jnp
from jax import lax
from jax.experimental import pallas as pl
from jax.experimental.pallas import tpu as pltpu
from jax.experimental.pallas import tpu_sc as plsc

N = 10000
E = 160000
D_IN = 256
D_OUT = 256
H = 128
NT = 16
RPT = 632
RPT_LAST = N - (NT - 1) * RPT

_mesh = plsc.VectorSubcoreMesh(core_axis_name="c", subcore_axis_name="s")


def _fill_ones(ref, n):
    one16 = jnp.full((16,), 1.0, jnp.float32)
    for j in range(n // 16):
        ref[pl.ds(j * 16, 16)] = one16
    if n % 16:
        ref[pl.ds(n - 16, 16)] = one16


CHA = 80
EPTA = E // NT
NCHA = EPTA // CHA
KA = 5
NGA = NCHA // KA


@functools.partial(
    pl.kernel,
    out_type=jax.ShapeDtypeStruct((2, N), jnp.float32),
    mesh=_mesh,
    scratch_types=[
        pltpu.VMEM((2, KA, CHA), jnp.int32),
        pltpu.VMEM((CHA,), jnp.float32),
        pltpu.VMEM_SHARED((N,), jnp.float32),
        pltpu.SemaphoreType.DMA((2,)),
        pltpu.SemaphoreType.DMA((2,)),
    ],
)
def _degree_kernel(src_hbm, dst_hbm, zeros_hbm, deg_hbm, idx_v, ones_v, acc_sh,
                   sem_i, sem_s):
    c = lax.axis_index("c")
    s = lax.axis_index("s")

    @pl.when(s == 0)
    def _():
        pltpu.sync_copy(zeros_hbm, acc_sh)

    _fill_ones(ones_v, CHA)
    plsc.subcore_barrier()

    base = s * EPTA

    def run(edge_hbm):
        def issue_group(g, p):
            for b in range(KA):
                off = base + g * (KA * CHA) + b * CHA
                pltpu.async_copy(edge_hbm.at[pl.ds(off, CHA)],
                                 idx_v.at[p, b], sem_i.at[p])

        def drain_scatters(p):
            for b in range(KA):
                pltpu.make_async_copy(ones_v, acc_sh.at[idx_v.at[p, b]],
                                      sem_s.at[p]).wait()

        issue_group(0, 0)

        def body(g, carry):
            p = lax.rem(g, 2)
            q = 1 - p

            @pl.when(g >= 1)
            def _():
                drain_scatters(q)

            @pl.when(g + 1 < NGA)
            def _():
                issue_group(g + 1, q)

            for b in range(KA):
                pltpu.make_async_copy(edge_hbm.at[pl.ds(base, CHA)],
                                      idx_v.at[p, b], sem_i.at[p]).wait()
            for b in range(KA):
                pltpu.async_copy(ones_v, acc_sh.at[idx_v.at[p, b]],
                                 sem_s.at[p], add=True)
            return carry

        lax.fori_loop(0, NGA, body, 0)
        drain_scatters((NGA - 1) % 2)

    @pl.when(c == 0)
    def _():
        run(src_hbm)

    @pl.when(c == 1)
    def _():
        run(dst_hbm)

    plsc.subcore_barrier()

    @pl.when(s == 0)
    def _():
        pltpu.sync_copy(acc_sh, deg_hbm.at[c])


def _scale_body(x_ref, degt_ref, gl_ref, gr_ref):
    dt = degt_ref[...]
    d2 = lax.rsqrt(jnp.maximum(dt[:, 0:1], 1.0))
    g = x_ref[...] * d2
    gl_ref[...] = g[:, :H]
    gr_ref[...] = g[:, H:]


def _scale_call(x, degT):
    R = 2000
    grid = N // R
    return pl.pallas_call(
        _scale_body,
        grid=(grid,),
        in_specs=[
            pl.BlockSpec((R, D_IN), lambda i: (i, 0)),
            pl.BlockSpec((R, 2), lambda i: (i, 0)),
        ],
        out_specs=[
            pl.BlockSpec((R, H), lambda i: (i, 0)),
            pl.BlockSpec((R, H), lambda i: (i, 0)),
        ],
        out_shape=[
            jax.ShapeDtypeStruct((N, H), jnp.float32),
            jax.ShapeDtypeStruct((N, H), jnp.float32),
        ],
    )(x, degT)


CHC = 80
EPT = E // NT
NGC = EPT // CHC
NSETS = 4


@functools.partial(
    pl.kernel,
    out_type=[
        jax.ShapeDtypeStruct((N, H), jnp.float32),
        jax.ShapeDtypeStruct((N, H), jnp.float32),
    ],
    mesh=_mesh,
    scratch_types=[
        pltpu.VMEM((NSETS, CHC), jnp.int32),
        pltpu.VMEM((NSETS, CHC), jnp.int32),
        pltpu.VMEM((NSETS, CHC, H), jnp.float32),
        pltpu.VMEM_SHARED((N, H), jnp.float32),
        pltpu.SemaphoreType.DMA((NSETS,)),
        pltpu.SemaphoreType.DMA((NSETS,)),
        pltpu.SemaphoreType.DMA((NSETS,)),
    ],
)
def _gather_scatter_kernel(gl_hbm, gr_hbm, src_hbm, dst_hbm, zeros_hbm,
                           aggl_hbm, aggr_hbm,
                           src_v, dst_v, rows_v, acc_sh,
                           sem_i, sem_g, sem_s):
    c = lax.axis_index("c")
    s = lax.axis_index("s")

    def _rowslice(ref):
        start = pl.multiple_of(s * RPT, 8)
        return ref.at[pl.ds(start, RPT)]

    def _rowslice_last(ref):
        return ref.at[pl.ds((NT - 1) * RPT, RPT_LAST)]

    @pl.when(s < NT - 1)
    def _():
        pltpu.sync_copy(_rowslice(zeros_hbm), _rowslice(acc_sh))

    @pl.when(s == NT - 1)
    def _():
        pltpu.sync_copy(_rowslice_last(zeros_hbm), _rowslice_last(acc_sh))

    plsc.subcore_barrier()

    base = s * EPT

    def run(g_hbm):
        def issue_idx(g):
            p = lax.rem(g, NSETS)
            off = base + g * CHC
            pltpu.async_copy(src_hbm.at[pl.ds(off, CHC)], src_v.at[p],
                             sem_i.at[p])
            pltpu.async_copy(dst_hbm.at[pl.ds(off, CHC)], dst_v.at[p],
                             sem_i.at[p])

        def wait_idx(g):
            p = lax.rem(g, NSETS)
            pltpu.make_async_copy(src_hbm.at[pl.ds(base, CHC)], src_v.at[p],
                                  sem_i.at[p]).wait()
            pltpu.make_async_copy(dst_hbm.at[pl.ds(base, CHC)], dst_v.at[p],
                                  sem_i.at[p]).wait()

        def issue_gather(g):
            p = lax.rem(g, NSETS)
            pltpu.async_copy(g_hbm.at[src_v.at[p]], rows_v.at[p], sem_g.at[p])

        def wait_gather(g):
            p = lax.rem(g, NSETS)
            pltpu.make_async_copy(g_hbm.at[src_v.at[p]], rows_v.at[p],
                                  sem_g.at[p]).wait()

        def issue_scatter(g):
            p = lax.rem(g, NSETS)
            pltpu.async_copy(rows_v.at[p], acc_sh.at[dst_v.at[p]],
                             sem_s.at[p], add=True)

        def drain_scatter(g):
            p = lax.rem(g, NSETS)
            pltpu.make_async_copy(rows_v.at[p], acc_sh.at[dst_v.at[p]],
                                  sem_s.at[p]).wait()

        issue_idx(0)
        issue_idx(1)
        wait_idx(0)
        issue_gather(0)

        def body(g, carry):
            @pl.when(g >= 2)
            def _():
                drain_scatter(g - 2)

            @pl.when(g + 2 < NGC)
            def _():
                issue_idx(g + 2)

            @pl.when(g + 1 < NGC)
            def _():
                wait_idx(g + 1)
                issue_gather(g + 1)

            wait_gather(g)
            issue_scatter(g)
            return carry

        lax.fori_loop(0, NGC, body, 0)
        drain_scatter(NGC - 2)
        drain_scatter(NGC - 1)

    @pl.when(c == 0)
    def _():
        run(gl_hbm)

    @pl.when(c == 1)
    def _():
        run(gr_hbm)

    plsc.subcore_barrier()

    def writeout(agg_hbm):
        @pl.when(s < NT - 1)
        def _():
            pltpu.sync_copy(_rowslice(acc_sh), _rowslice(agg_hbm))

        @pl.when(s == NT - 1)
        def _():
            pltpu.sync_copy(_rowslice_last(acc_sh), _rowslice_last(agg_hbm))

    @pl.when(c == 0)
    def _():
        writeout(aggl_hbm)

    @pl.when(c == 1)
    def _():
        writeout(aggr_hbm)


def _matmul_body(al_ref, ar_ref, w_ref, ind_ref, ft_ref):
    w = w_ref[...]
    acc = lax.dot_general(al_ref[...], w[:, :H], (((1,), (1,)), ((), ())),
                          preferred_element_type=jnp.float32)
    acc = acc + lax.dot_general(ar_ref[...], w[:, H:], (((1,), (1,)), ((), ())),
                                preferred_element_type=jnp.float32)
    d0 = lax.rsqrt(jnp.maximum(ind_ref[...], 1.0))
    ft_ref[...] = acc * d0


def _matmul_call(aggl, aggr, W_fc, ind):
    R = 2000
    grid = N // R
    return pl.pallas_call(
        _matmul_body,
        grid=(grid,),
        in_specs=[
            pl.BlockSpec((R, H), lambda i: (i, 0)),
            pl.BlockSpec((R, H), lambda i: (i, 0)),
            pl.BlockSpec((D_OUT, D_IN), lambda i: (0, 0)),
            pl.BlockSpec((R, 1), lambda i: (i, 0)),
        ],
        out_specs=pl.BlockSpec((R, D_OUT), lambda i: (i, 0)),
        out_shape=jax.ShapeDtypeStruct((N, D_OUT), jnp.float32),
    )(aggl, aggr, W_fc, ind)


def kernel(x, loc, edge_index, inter_ids, W_fc, W_G, embed_table, boundaries):
    src = edge_index[0]
    dst = edge_index[1]
    zeros_n = jnp.zeros((N,), jnp.float32)
    zeros_nd = jnp.zeros((N, H), jnp.float32)

    deg = _degree_kernel(src, dst, zeros_n)
    gl, gr = _scale_call(x, deg.T)
    aggl, aggr = _gather_scatter_kernel(gl, gr, src, dst, zeros_nd)
    return _matmul_call(aggl, aggr, W_fc, deg[1].reshape(N, 1))


# speedup vs baseline: 1.1646x; 1.0043x over previous
import functools

import jax
import jax.numpy as jnp
from jax import lax
from jax.experimental import pallas as pl
from jax.experimental.pallas import tpu as pltpu
from jax.experimental.pallas import tpu_sc as plsc

N = 10000
E = 160000
D_IN = 256
D_OUT = 256
H = 128
NT = 16
RPT = 632
RPT_LAST = N - (NT - 1) * RPT

_mesh = plsc.VectorSubcoreMesh(core_axis_name="c", subcore_axis_name="s")


def _fill_ones(ref, n):
    one16 = jnp.full((16,), 1.0, jnp.float32)
    for j in range(n // 16):
        ref[pl.ds(j * 16, 16)] = one16
    if n % 16:
        ref[pl.ds(n - 16, 16)] = one16


CHA = 80
EPTA = E // NT
NCHA = EPTA // CHA
KA = 5
NGA = NCHA // KA


@functools.partial(
    pl.kernel,
    out_type=jax.ShapeDtypeStruct((2, N), jnp.float32),
    mesh=_mesh,
    scratch_types=[
        pltpu.VMEM((2, KA, CHA), jnp.int32),
        pltpu.VMEM((CHA,), jnp.float32),
        pltpu.VMEM_SHARED((N,), jnp.float32),
        pltpu.SemaphoreType.DMA((2,)),
        pltpu.SemaphoreType.DMA((2,)),
    ],
)
def _degree_kernel(src_hbm, dst_hbm, zeros_hbm, deg_hbm, idx_v, ones_v, acc_sh,
                   sem_i, sem_s):
    c = lax.axis_index("c")
    s = lax.axis_index("s")

    @pl.when(s == 0)
    def _():
        pltpu.sync_copy(zeros_hbm, acc_sh)

    _fill_ones(ones_v, CHA)
    plsc.subcore_barrier()

    base = s * EPTA

    def run(edge_hbm):
        def issue_group(g, p):
            for b in range(KA):
                off = base + g * (KA * CHA) + b * CHA
                pltpu.async_copy(edge_hbm.at[pl.ds(off, CHA)],
                                 idx_v.at[p, b], sem_i.at[p])

        def drain_scatters(p):
            for b in range(KA):
                pltpu.make_async_copy(ones_v, acc_sh.at[idx_v.at[p, b]],
                                      sem_s.at[p]).wait()

        issue_group(0, 0)

        def body(g, carry):
            p = lax.rem(g, 2)
            q = 1 - p

            @pl.when(g >= 1)
            def _():
                drain_scatters(q)

            @pl.when(g + 1 < NGA)
            def _():
                issue_group(g + 1, q)

            for b in range(KA):
                pltpu.make_async_copy(edge_hbm.at[pl.ds(base, CHA)],
                                      idx_v.at[p, b], sem_i.at[p]).wait()
            for b in range(KA):
                pltpu.async_copy(ones_v, acc_sh.at[idx_v.at[p, b]],
                                 sem_s.at[p], add=True)
            return carry

        lax.fori_loop(0, NGA, body, 0)
        drain_scatters((NGA - 1) % 2)

    @pl.when(c == 0)
    def _():
        run(src_hbm)

    @pl.when(c == 1)
    def _():
        run(dst_hbm)

    plsc.subcore_barrier()

    @pl.when(s == 0)
    def _():
        pltpu.sync_copy(acc_sh, deg_hbm.at[c])


def _scale_body(x_ref, degt_ref, gl_ref, gr_ref):
    dt = degt_ref[...]
    d2 = lax.rsqrt(jnp.maximum(dt[:, 0:1], 1.0))
    g = x_ref[...] * d2
    gl_ref[...] = g[:, :H]
    gr_ref[...] = g[:, H:]


def _scale_call(x, degT):
    R = 2000
    grid = N // R
    return pl.pallas_call(
        _scale_body,
        grid=(grid,),
        in_specs=[
            pl.BlockSpec((R, D_IN), lambda i: (i, 0)),
            pl.BlockSpec((R, 2), lambda i: (i, 0)),
        ],
        out_specs=[
            pl.BlockSpec((R, H), lambda i: (i, 0)),
            pl.BlockSpec((R, H), lambda i: (i, 0)),
        ],
        out_shape=[
            jax.ShapeDtypeStruct((N, H), jnp.float32),
            jax.ShapeDtypeStruct((N, H), jnp.float32),
        ],
    )(x, degT)


CHC = 80
EPT = E // NT
NGC = EPT // CHC
NSETS = 4
NIDX = 5


@functools.partial(
    pl.kernel,
    out_type=[
        jax.ShapeDtypeStruct((N, H), jnp.float32),
        jax.ShapeDtypeStruct((N, H), jnp.float32),
    ],
    mesh=_mesh,
    scratch_types=[
        pltpu.VMEM((NIDX, CHC), jnp.int32),
        pltpu.VMEM((NIDX, CHC), jnp.int32),
        pltpu.VMEM((NSETS, CHC, H), jnp.float32),
        pltpu.VMEM_SHARED((N, H), jnp.float32),
        pltpu.SemaphoreType.DMA((NIDX,)),
        pltpu.SemaphoreType.DMA((NSETS,)),
        pltpu.SemaphoreType.DMA((NSETS,)),
    ],
)
def _gather_scatter_kernel(gl_hbm, gr_hbm, src_hbm, dst_hbm, zeros_hbm,
                           aggl_hbm, aggr_hbm,
                           src_v, dst_v, rows_v, acc_sh,
                           sem_i, sem_g, sem_s):
    c = lax.axis_index("c")
    s = lax.axis_index("s")

    def _rowslice(ref):
        start = pl.multiple_of(s * RPT, 8)
        return ref.at[pl.ds(start, RPT)]

    def _rowslice_last(ref):
        return ref.at[pl.ds((NT - 1) * RPT, RPT_LAST)]

    @pl.when(s < NT - 1)
    def _():
        pltpu.sync_copy(_rowslice(zeros_hbm), _rowslice(acc_sh))

    @pl.when(s == NT - 1)
    def _():
        pltpu.sync_copy(_rowslice_last(zeros_hbm), _rowslice_last(acc_sh))

    plsc.subcore_barrier()

    base = s * EPT

    def run(g_hbm):
        def issue_idx(g):
            p = lax.rem(g, NIDX)
            off = base + g * CHC
            pltpu.async_copy(src_hbm.at[pl.ds(off, CHC)], src_v.at[p],
                             sem_i.at[p])
            pltpu.async_copy(dst_hbm.at[pl.ds(off, CHC)], dst_v.at[p],
                             sem_i.at[p])

        def wait_idx(g):
            p = lax.rem(g, NIDX)
            pltpu.make_async_copy(src_hbm.at[pl.ds(base, CHC)], src_v.at[p],
                                  sem_i.at[p]).wait()
            pltpu.make_async_copy(dst_hbm.at[pl.ds(base, CHC)], dst_v.at[p],
                                  sem_i.at[p]).wait()

        def issue_gather(g):
            pltpu.async_copy(g_hbm.at[src_v.at[lax.rem(g, NIDX)]],
                             rows_v.at[lax.rem(g, NSETS)],
                             sem_g.at[lax.rem(g, NSETS)])

        def wait_gather(g):
            pltpu.make_async_copy(g_hbm.at[src_v.at[lax.rem(g, NIDX)]],
                                  rows_v.at[lax.rem(g, NSETS)],
                                  sem_g.at[lax.rem(g, NSETS)]).wait()

        def issue_scatter(g):
            pltpu.async_copy(rows_v.at[lax.rem(g, NSETS)],
                             acc_sh.at[dst_v.at[lax.rem(g, NIDX)]],
                             sem_s.at[lax.rem(g, NSETS)], add=True)

        def drain_scatter(g):
            pltpu.make_async_copy(rows_v.at[lax.rem(g, NSETS)],
                                  acc_sh.at[dst_v.at[lax.rem(g, NIDX)]],
                                  sem_s.at[lax.rem(g, NSETS)]).wait()

        issue_idx(0)
        issue_idx(1)
        issue_idx(2)
        wait_idx(0)
        issue_gather(0)
        wait_idx(1)
        issue_gather(1)

        def body(g, carry):
            @pl.when(g >= 2)
            def _():
                drain_scatter(g - 2)

            @pl.when(g + 3 < NGC)
            def _():
                issue_idx(g + 3)

            @pl.when(g + 2 < NGC)
            def _():
                wait_idx(g + 2)
                issue_gather(g + 2)

            wait_gather(g)
            issue_scatter(g)
            return carry

        lax.fori_loop(0, NGC, body, 0)
        drain_scatter(NGC - 2)
        drain_scatter(NGC - 1)

    @pl.when(c == 0)
    def _():
        run(gl_hbm)

    @pl.when(c == 1)
    def _():
        run(gr_hbm)

    plsc.subcore_barrier()

    def writeout(agg_hbm):
        @pl.when(s < NT - 1)
        def _():
            pltpu.sync_copy(_rowslice(acc_sh), _rowslice(agg_hbm))

        @pl.when(s == NT - 1)
        def _():
            pltpu.sync_copy(_rowslice_last(acc_sh), _rowslice_last(agg_hbm))

    @pl.when(c == 0)
    def _():
        writeout(aggl_hbm)

    @pl.when(c == 1)
    def _():
        writeout(aggr_hbm)


def _matmul_body(al_ref, ar_ref, w_ref, ind_ref, ft_ref):
    w = w_ref[...]
    acc = lax.dot_general(al_ref[...], w[:, :H], (((1,), (1,)), ((), ())),
                          preferred_element_type=jnp.float32)
    acc = acc + lax.dot_general(ar_ref[...], w[:, H:], (((1,), (1,)), ((), ())),
                                preferred_element_type=jnp.float32)
    d0 = lax.rsqrt(jnp.maximum(ind_ref[...], 1.0))
    ft_ref[...] = acc * d0


def _matmul_call(aggl, aggr, W_fc, ind):
    R = 2000
    grid = N // R
    return pl.pallas_call(
        _matmul_body,
        grid=(grid,),
        in_specs=[
            pl.BlockSpec((R, H), lambda i: (i, 0)),
            pl.BlockSpec((R, H), lambda i: (i, 0)),
            pl.BlockSpec((D_OUT, D_IN), lambda i: (0, 0)),
            pl.BlockSpec((R, 1), lambda i: (i, 0)),
        ],
        out_specs=pl.BlockSpec((R, D_OUT), lambda i: (i, 0)),
        out_shape=jax.ShapeDtypeStruct((N, D_OUT), jnp.float32),
    )(aggl, aggr, W_fc, ind)


def kernel(x, loc, edge_index, inter_ids, W_fc, W_G, embed_table, boundaries):
    src = edge_index[0]
    dst = edge_index[1]
    zeros_n = jnp.zeros((N,), jnp.float32)
    zeros_nd = jnp.zeros((N, H), jnp.float32)

    deg = _degree_kernel(src, dst, zeros_n)
    gl, gr = _scale_call(x, deg.T)
    aggl, aggr = _gather_scatter_kernel(gl, gr, src, dst, zeros_nd)
    return _matmul_call(aggl, aggr, W_fc, deg[1].reshape(N, 1))
